# baseline (speedup 1.0000x reference)
import functools

import jax
import jax.numpy as jnp
from jax import lax
from jax.experimental import pallas as pl
from jax.experimental.pallas import tpu as pltpu

N_IN = 20000
N_PAD = 20480
PRE_NMS = 6000
C_PAD = 6144
PROPOSAL_COUNT = 1000
OUT_PAD = 1024
NMS_THRESHOLD = 0.7
IMG_H = 1024.0
IMG_W = 1024.0

_BLK = 256
_NBLK = C_PAD // _BLK
_JCH = 1024
_NJCH = C_PAD // _JCH
_ITILE = 256
_JTILE = 2048
_KCH = 512

_HI = lax.Precision.HIGHEST


_ROWS = N_PAD // 128


def _k1_select(s2d_ref, msel_ref, pose_ref):
    u = lax.bitcast_convert_type(s2d_ref[...], jnp.int32)

    tri128 = jnp.where(
        lax.broadcasted_iota(jnp.int32, (128, 128), 0)
        <= lax.broadcasted_iota(jnp.int32, (128, 128), 1),
        1.0, 0.0).astype(jnp.float32)
    tris = jnp.where(
        lax.broadcasted_iota(jnp.int32, (_ROWS, _ROWS), 1)
        < lax.broadcasted_iota(jnp.int32, (_ROWS, _ROWS), 0),
        1.0, 0.0).astype(jnp.float32)

    def excl_cumsum(m):
        incl = lax.dot_general(m, tri128, (((1,), (0,)), ((), ())),
                               preferred_element_type=jnp.float32)
        rowsum = incl[:, 127:128]
        rowpref = lax.dot_general(tris, rowsum, (((1,), (0,)), ((), ())),
                                  preferred_element_type=jnp.float32)
        return incl + rowpref - m

    def bs(_, carry):
        lo, hi = carry
        mid = (lo + hi + 1) >> 1
        cnt = jnp.sum(jnp.where(u >= mid, 1.0, 0.0))
        ok = cnt >= float(PRE_NMS)
        return (jnp.where(ok, mid, lo), jnp.where(ok, hi, mid - 1))

    lo, _ = lax.fori_loop(0, 30, bs,
                          (jnp.int32(0), jnp.int32(0x3F800000)))
    t_val = lo
    gt = u > t_val
    n_gt = jnp.sum(jnp.where(gt, 1.0, 0.0))
    m_eq = jnp.where(u == t_val, 1.0, 0.0)
    quota = float(PRE_NMS) - n_gt
    eqex = excl_cumsum(m_eq)
    tie_sel = (m_eq > 0.5) & (eqex < quota)
    msel = jnp.where(gt | tie_sel, 1.0, 0.0)
    msel_ref[...] = msel
    pose_ref[...] = excl_cumsum(msel)


def _k2_gather_decode(msel_row_ref, pose_row_ref, vals_ref, out_ref,
                      dec_ref, srow_ref, rrow_ref):
    pio = lax.broadcasted_iota(jnp.int32, (C_PAD, 1), 0).astype(jnp.float32)

    i256 = jnp.where(
        lax.broadcasted_iota(jnp.int32, (_BLK, _BLK), 0)
        == lax.broadcasted_iota(jnp.int32, (_BLK, _BLK), 1),
        1.0, 0.0).astype(jnp.float32)

    def row_from_col(v):
        return lax.dot_general(v, i256, (((0,), (0,)), ((), ())),
                               precision=_HI,
                               preferred_element_type=jnp.float32)

    def body(jc, acc):
        p = pose_row_ref[:, pl.ds(jc * _KCH, _KCH)]
        m = msel_row_ref[:, pl.ds(jc * _KCH, _KCH)]
        oh = jnp.where((m > 0.5) & (p == pio), 1.0, 0.0)
        v = vals_ref[pl.ds(jc * _KCH, _KCH), :]
        return acc + lax.dot_general(
            oh, v, (((1,), (0,)), ((), ())),
            precision=_HI, preferred_element_type=jnp.float32)

    acc = lax.fori_loop(0, N_PAD // _KCH, body,
                        jnp.zeros((C_PAD, 16), jnp.float32))

    d0 = acc[:, 0:1]
    d1 = acc[:, 1:2]
    d2 = acc[:, 2:3]
    d3 = acc[:, 3:4]
    ay1 = acc[:, 4:5]
    ax1 = acc[:, 5:6]
    ay2 = acc[:, 6:7]
    ax2 = acc[:, 7:8]
    sc = acc[:, 8:9]

    height = ay2 - ay1
    width = ax2 - ax1
    center_y = ay1 + 0.5 * height
    center_x = ax1 + 0.5 * width
    center_y = center_y + d0 * height
    center_x = center_x + d1 * width
    height = height * jnp.exp(d2)
    width = width * jnp.exp(d3)
    y1 = center_y - 0.5 * height
    x1 = center_x - 0.5 * width
    y2 = y1 + height
    x2 = x1 + width
    y1 = jnp.clip(y1, 0.0, IMG_H)
    x1 = jnp.clip(x1, 0.0, IMG_W)
    y2 = jnp.clip(y2, 0.0, IMG_H)
    x2 = jnp.clip(x2, 0.0, IMG_W)
    area = jnp.maximum(y2 - y1, 0.0) * jnp.maximum(x2 - x1, 0.0)

    z = jnp.zeros((C_PAD, 1), jnp.float32)
    dec_ref[...] = jnp.concatenate(
        [y1, x1, y2, x2, area, sc, z, z, z, z, z, z, z, z, z, z], axis=1)

    def trow(t, _):
        scol = dec_ref[pl.ds(t * _BLK, _BLK), 5:6]
        srow_ref[0:1, pl.ds(t * _BLK, _BLK)] = row_from_col(scol)
        return 0

    lax.fori_loop(0, _NBLK, trow, 0)

    def rank_tile(t, _):
        si = dec_ref[pl.ds(t * _BLK, _BLK), 5:6]
        ii = t * _BLK + lax.broadcasted_iota(jnp.int32, (_BLK, 1), 0)

        def inner(c, cnt):
            sj = srow_ref[:, pl.ds(c * _JCH, _JCH)]
            jj = c * _JCH + lax.broadcasted_iota(jnp.int32, (1, _JCH), 1)
            cmp = (sj > si) | ((sj == si) & (jj < ii))
            return cnt + jnp.sum(jnp.where(cmp, 1.0, 0.0),
                                 axis=1, keepdims=True)

        cnt = lax.fori_loop(0, _NJCH, inner,
                            jnp.zeros((_BLK, 1), jnp.float32))
        rrow_ref[0:1, pl.ds(t * _BLK, _BLK)] = row_from_col(cnt)
        return 0

    lax.fori_loop(0, _NBLK, rank_tile, 0)

    def reorder(jc, acc2):
        r = rrow_ref[:, pl.ds(jc * _KCH, _KCH)]
        oh2 = jnp.where(r == pio, 1.0, 0.0)
        v = dec_ref[pl.ds(jc * _KCH, _KCH), :]
        return acc2 + lax.dot_general(
            oh2, v, (((1,), (0,)), ((), ())),
            precision=_HI, preferred_element_type=jnp.float32)

    out_ref[...] = lax.fori_loop(0, C_PAD // _KCH, reorder,
                                 jnp.zeros((C_PAD, 16), jnp.float32))


def _k3_nms_assemble(bc_ref, br_ref, out_ref, kept_ref, pos_ref):
    i256 = jnp.where(
        lax.broadcasted_iota(jnp.int32, (_BLK, _BLK), 0)
        == lax.broadcasted_iota(jnp.int32, (_BLK, _BLK), 1),
        1.0, 0.0).astype(jnp.float32)
    ltm = jnp.where(
        lax.broadcasted_iota(jnp.int32, (_BLK, _BLK), 0)
        < lax.broadcasted_iota(jnp.int32, (_BLK, _BLK), 1),
        1.0, 0.0).astype(jnp.float32)
    tri = jnp.where(
        lax.broadcasted_iota(jnp.int32, (_BLK, _BLK), 0)
        <= lax.broadcasted_iota(jnp.int32, (_BLK, _BLK), 1),
        1.0, 0.0).astype(jnp.float32)

    def row_from_col(v):
        return lax.dot_general(v, i256, (((0,), (0,)), ((), ())),
                               preferred_element_type=jnp.float32)

    def col_from_row(v):
        return lax.dot_general(i256, v, (((1,), (1,)), ((), ())),
                               preferred_element_type=jnp.float32)

    kept_ref[...] = jnp.zeros((1, C_PAD), jnp.float32)

    def block(t, _):
        base = t * _BLK
        y1b = bc_ref[pl.ds(base, _BLK), 0:1]
        x1b = bc_ref[pl.ds(base, _BLK), 1:2]
        y2b = bc_ref[pl.ds(base, _BLK), 2:3]
        x2b = bc_ref[pl.ds(base, _BLK), 3:4]
        area_b = jnp.maximum(y2b - y1b, 0.0) * jnp.maximum(x2b - x1b, 0.0)

        def chunk(c, sup):
            sl = pl.ds(c * _JCH, _JCH)
            y1c = br_ref[0:1, sl]
            x1c = br_ref[1:2, sl]
            y2c = br_ref[2:3, sl]
            x2c = br_ref[3:4, sl]
            area_c = br_ref[4:5, sl]
            kc = kept_ref[0:1, sl]
            yy1 = jnp.maximum(y1b, y1c)
            xx1 = jnp.maximum(x1b, x1c)
            yy2 = jnp.minimum(y2b, y2c)
            xx2 = jnp.minimum(x2b, x2c)
            inter = jnp.maximum(yy2 - yy1, 0.0) * jnp.maximum(xx2 - xx1, 0.0)
            union = area_b + area_c - inter
            ious = inter / (union + 1e-9)
            hit = jnp.where((ious > NMS_THRESHOLD) & (kc > 0.5), 1.0, 0.0)
            return jnp.maximum(sup, jnp.max(hit, axis=1, keepdims=True))

        sup_col = lax.fori_loop(0, _NJCH, chunk,
                                jnp.zeros((_BLK, 1), jnp.float32))

        sb = pl.ds(base, _BLK)
        y1s = br_ref[0:1, sb]
        x1s = br_ref[1:2, sb]
        y2s = br_ref[2:3, sb]
        x2s = br_ref[3:4, sb]
        area_s = br_ref[4:5, sb]
        yy1 = jnp.maximum(y1b, y1s)
        xx1 = jnp.maximum(x1b, x1s)
        yy2 = jnp.minimum(y2b, y2s)
        xx2 = jnp.minimum(x2b, x2s)
        inter = jnp.maximum(yy2 - yy1, 0.0) * jnp.maximum(xx2 - xx1, 0.0)
        union = area_b + area_s - inter
        ious = inter / (union + 1e-9)
        of = jnp.where(ious > NMS_THRESHOLD, 1.0, 0.0) * ltm

        jrow = base + lax.broadcasted_iota(jnp.int32, (1, _BLK), 1)
        invalid = jnp.where(jrow >= PRE_NMS, 1.0, 0.0)
        sup0 = jnp.maximum(row_from_col(sup_col), invalid)
        dec0 = jnp.zeros((1, _BLK), jnp.float32)

        def cond(state):
            sup, dec = state
            return jnp.sum((1.0 - sup) * (1.0 - dec)) > 0.5

        def wave(state):
            sup, dec = state
            und = (1.0 - sup) * (1.0 - dec)
            undc = col_from_row(und)
            blocked = jnp.max(of * undc, axis=0, keepdims=True)
            newk = und * (1.0 - blocked)
            dec = jnp.maximum(dec, newk)
            decc = col_from_row(dec)
            supn = jnp.max(of * decc, axis=0, keepdims=True)
            sup = jnp.maximum(sup, supn * (1.0 - dec))
            return (sup, dec)

        _, dec = lax.while_loop(cond, wave, (sup0, dec0))
        kept_ref[0:1, pl.ds(base, _BLK)] = dec
        return 0

    lax.fori_loop(0, _NBLK, block, 0)

    def csum(c, carry):
        kc = kept_ref[0:1, pl.ds(c * _BLK, _BLK)]
        inc = lax.dot_general(kc, tri, (((1,), (0,)), ((), ())),
                              preferred_element_type=jnp.float32)
        pos_ref[0:1, pl.ds(c * _BLK, _BLK)] = inc + carry
        return carry + jnp.sum(kc)

    nkept = lax.fori_loop(0, _NBLK, csum, jnp.float32(0.0))

    kept = kept_ref[...]
    pos_excl = pos_ref[...] - kept
    pio = lax.broadcasted_iota(jnp.int32, (OUT_PAD, 1), 0).astype(jnp.float32)
    jio = lax.broadcasted_iota(jnp.int32, (1, C_PAD), 1)
    eq = (kept > 0.5) & (pos_excl == pio)
    fb = (pio >= nkept) & (jio == 0)
    oh = jnp.where(eq | fb, 1.0, 0.0).astype(jnp.float32)
    res = lax.dot_general(oh, bc_ref[...], (((1,), (0,)), ((), ())),
                          precision=_HI, preferred_element_type=jnp.float32)
    scale = 1.0 / IMG_H
    z = jnp.zeros((OUT_PAD, 1), jnp.float32)
    out_ref[...] = jnp.concatenate(
        [res[:, 0:1] * scale, res[:, 1:2] * scale,
         res[:, 2:3] * scale, res[:, 3:4] * scale,
         res[:, 5:6], z, z, z, z, z, z, z, z, z, z, z], axis=1)


@functools.partial(jax.jit, static_argnames=("interpret",))
def _pipeline(rpn_pred_probs, rpn_box_deltas, anchors, interpret=False):
    scores = rpn_pred_probs[:, 1]
    pad = N_PAD - N_IN
    sp = jnp.concatenate([scores, jnp.full((pad,), -1.0, jnp.float32)])
    s2d = sp.reshape(_ROWS, 128)
    dz = jnp.zeros((pad, 4), jnp.float32)
    vals = jnp.concatenate(
        [jnp.concatenate([rpn_box_deltas, dz], axis=0),
         jnp.concatenate([anchors, dz], axis=0),
         sp.reshape(N_PAD, 1),
         jnp.zeros((N_PAD, 7), jnp.float32)], axis=1)

    msel, pose = pl.pallas_call(
        _k1_select,
        out_shape=[jax.ShapeDtypeStruct((_ROWS, 128), jnp.float32),
                   jax.ShapeDtypeStruct((_ROWS, 128), jnp.float32)],
        interpret=interpret,
    )(s2d)

    bc = pl.pallas_call(
        _k2_gather_decode,
        out_shape=jax.ShapeDtypeStruct((C_PAD, 16), jnp.float32),
        scratch_shapes=[pltpu.VMEM((C_PAD, 16), jnp.float32),
                        pltpu.VMEM((1, C_PAD), jnp.float32),
                        pltpu.VMEM((1, C_PAD), jnp.float32)],
        interpret=interpret,
    )(msel.reshape(1, N_PAD), pose.reshape(1, N_PAD), vals)

    br = bc.T

    res = pl.pallas_call(
        _k3_nms_assemble,
        out_shape=jax.ShapeDtypeStruct((OUT_PAD, 16), jnp.float32),
        scratch_shapes=[pltpu.VMEM((1, C_PAD), jnp.float32),
                        pltpu.VMEM((1, C_PAD), jnp.float32)],
        interpret=interpret,
    )(bc, br)

    return res[:PROPOSAL_COUNT, :5]


def kernel(rpn_pred_probs, rpn_box_deltas, anchors):
    return _pipeline(rpn_pred_probs, rpn_box_deltas, anchors)


# speedup vs baseline: 1.7192x; 1.7192x over previous
import functools

import jax
import jax.numpy as jnp
from jax import lax
from jax.experimental import pallas as pl
from jax.experimental.pallas import tpu as pltpu

N_IN = 20000
N_PAD = 20480
PRE_NMS = 6000
C_PAD = 6144
PROPOSAL_COUNT = 1000
OUT_PAD = 1024
NMS_THRESHOLD = 0.7
IMG_H = 1024.0
IMG_W = 1024.0

_BLK = 256
_NBLK = C_PAD // _BLK
_JCH = 1024
_NJCH = C_PAD // _JCH
_ITILE = 256
_JTILE = 2048
_KCH = 512
_WIN = 1024

_HI = lax.Precision.HIGHEST


_ROWS = N_PAD // 128


def _k1_select(s2d_ref, msel_ref, pose_ref, bases_ref):
    u = lax.bitcast_convert_type(s2d_ref[...], jnp.int32)

    tri128 = jnp.where(
        lax.broadcasted_iota(jnp.int32, (128, 128), 0)
        <= lax.broadcasted_iota(jnp.int32, (128, 128), 1),
        1.0, 0.0).astype(jnp.float32)
    tris = jnp.where(
        lax.broadcasted_iota(jnp.int32, (_ROWS, _ROWS), 1)
        < lax.broadcasted_iota(jnp.int32, (_ROWS, _ROWS), 0),
        1.0, 0.0).astype(jnp.float32)

    def excl_cumsum(m):
        incl = lax.dot_general(m, tri128, (((1,), (0,)), ((), ())),
                               preferred_element_type=jnp.float32)
        rowsum = incl[:, 127:128]
        rowpref = lax.dot_general(tris, rowsum, (((1,), (0,)), ((), ())),
                                  preferred_element_type=jnp.float32)
        return incl + rowpref - m

    def bs(_, carry):
        lo, hi = carry
        mid = (lo + hi + 1) >> 1
        cnt = jnp.sum(jnp.where(u >= mid, 1.0, 0.0))
        ok = cnt >= float(PRE_NMS)
        return (jnp.where(ok, mid, lo), jnp.where(ok, hi, mid - 1))

    lo, _ = lax.fori_loop(0, 30, bs,
                          (jnp.int32(0), jnp.int32(0x3F800000)))
    t_val = lo
    gt = u > t_val
    n_gt = jnp.sum(jnp.where(gt, 1.0, 0.0))
    m_eq = jnp.where(u == t_val, 1.0, 0.0)
    quota = float(PRE_NMS) - n_gt
    eqex = excl_cumsum(m_eq)
    tie_sel = (m_eq > 0.5) & (eqex < quota)
    msel = jnp.where(gt | tie_sel, 1.0, 0.0)
    msel_ref[...] = msel
    pose_ref[...] = excl_cumsum(msel)

    ones128 = jnp.ones((128, 1), jnp.float32)
    rowsums = lax.dot_general(msel, ones128, (((1,), (0,)), ((), ())),
                              preferred_element_type=jnp.float32)
    gmask = jnp.where(
        lax.broadcasted_iota(jnp.int32, (N_PAD // _KCH, _ROWS), 1)
        < 4 * lax.broadcasted_iota(jnp.int32, (N_PAD // _KCH, _ROWS), 0),
        1.0, 0.0).astype(jnp.float32)
    bases_ref[...] = lax.dot_general(gmask, rowsums,
                                     (((1,), (0,)), ((), ())),
                                     preferred_element_type=jnp.float32)


def _k2_gather_decode(bases_ref, msel_row_ref, pose_row_ref, vals_ref, out_ref,
                      acc_ref, dec_ref, srow_ref, rrow_ref):
    pio = lax.broadcasted_iota(jnp.int32, (C_PAD, 1), 0).astype(jnp.float32)

    i256 = jnp.where(
        lax.broadcasted_iota(jnp.int32, (_BLK, _BLK), 0)
        == lax.broadcasted_iota(jnp.int32, (_BLK, _BLK), 1),
        1.0, 0.0).astype(jnp.float32)

    def row_from_col(v):
        return lax.dot_general(v, i256, (((0,), (0,)), ((), ())),
                               precision=_HI,
                               preferred_element_type=jnp.float32)

    acc_ref[...] = jnp.zeros((C_PAD + _WIN, 16), jnp.float32)
    wio = lax.broadcasted_iota(jnp.int32, (_WIN, 1), 0)

    def body(jc, _):
        dbase = pl.multiple_of(bases_ref[jc], 8)
        p = pose_row_ref[:, pl.ds(jc * _KCH, _KCH)]
        m = msel_row_ref[:, pl.ds(jc * _KCH, _KCH)]
        piow = (dbase + wio).astype(jnp.float32)
        oh = jnp.where((m > 0.5) & (p == piow), 1.0, 0.0)
        v = vals_ref[pl.ds(jc * _KCH, _KCH), :]
        win = acc_ref[pl.ds(dbase, _WIN), :]
        acc_ref[pl.ds(dbase, _WIN), :] = win + lax.dot_general(
            oh, v, (((1,), (0,)), ((), ())),
            precision=_HI, preferred_element_type=jnp.float32)
        return 0

    lax.fori_loop(0, N_PAD // _KCH, body, 0)
    acc = acc_ref[0:C_PAD, :]

    d0 = acc[:, 0:1]
    d1 = acc[:, 1:2]
    d2 = acc[:, 2:3]
    d3 = acc[:, 3:4]
    ay1 = acc[:, 4:5]
    ax1 = acc[:, 5:6]
    ay2 = acc[:, 6:7]
    ax2 = acc[:, 7:8]
    sc = acc[:, 8:9]

    height = ay2 - ay1
    width = ax2 - ax1
    center_y = ay1 + 0.5 * height
    center_x = ax1 + 0.5 * width
    center_y = center_y + d0 * height
    center_x = center_x + d1 * width
    height = height * jnp.exp(d2)
    width = width * jnp.exp(d3)
    y1 = center_y - 0.5 * height
    x1 = center_x - 0.5 * width
    y2 = y1 + height
    x2 = x1 + width
    y1 = jnp.clip(y1, 0.0, IMG_H)
    x1 = jnp.clip(x1, 0.0, IMG_W)
    y2 = jnp.clip(y2, 0.0, IMG_H)
    x2 = jnp.clip(x2, 0.0, IMG_W)
    area = jnp.maximum(y2 - y1, 0.0) * jnp.maximum(x2 - x1, 0.0)

    z = jnp.zeros((C_PAD, 1), jnp.float32)
    dec_ref[...] = jnp.concatenate(
        [y1, x1, y2, x2, area, sc, z, z, z, z, z, z, z, z, z, z], axis=1)

    def trow(t, _):
        scol = dec_ref[pl.ds(t * _BLK, _BLK), 5:6]
        srow_ref[0:1, pl.ds(t * _BLK, _BLK)] = row_from_col(scol)
        return 0

    lax.fori_loop(0, _NBLK, trow, 0)

    def rank_tile(t, _):
        si = dec_ref[pl.ds(t * _BLK, _BLK), 5:6]
        ii = t * _BLK + lax.broadcasted_iota(jnp.int32, (_BLK, 1), 0)

        def inner(c, cnt):
            sj = srow_ref[:, pl.ds(c * _JCH, _JCH)]
            jj = c * _JCH + lax.broadcasted_iota(jnp.int32, (1, _JCH), 1)
            cmp = (sj > si) | ((sj == si) & (jj < ii))
            return cnt + jnp.sum(jnp.where(cmp, 1.0, 0.0),
                                 axis=1, keepdims=True)

        cnt = lax.fori_loop(0, _NJCH, inner,
                            jnp.zeros((_BLK, 1), jnp.float32))
        rrow_ref[0:1, pl.ds(t * _BLK, _BLK)] = row_from_col(cnt)
        return 0

    lax.fori_loop(0, _NBLK, rank_tile, 0)

    def reorder(jc, acc2):
        r = rrow_ref[:, pl.ds(jc * _KCH, _KCH)]
        oh2 = jnp.where(r == pio, 1.0, 0.0)
        v = dec_ref[pl.ds(jc * _KCH, _KCH), :]
        return acc2 + lax.dot_general(
            oh2, v, (((1,), (0,)), ((), ())),
            precision=_HI, preferred_element_type=jnp.float32)

    out_ref[...] = lax.fori_loop(0, C_PAD // _KCH, reorder,
                                 jnp.zeros((C_PAD, 16), jnp.float32))


def _k3_nms_assemble(bc_ref, br_ref, out_ref, kept_ref, pos_ref):
    i256 = jnp.where(
        lax.broadcasted_iota(jnp.int32, (_BLK, _BLK), 0)
        == lax.broadcasted_iota(jnp.int32, (_BLK, _BLK), 1),
        1.0, 0.0).astype(jnp.float32)
    ltm = jnp.where(
        lax.broadcasted_iota(jnp.int32, (_BLK, _BLK), 0)
        < lax.broadcasted_iota(jnp.int32, (_BLK, _BLK), 1),
        1.0, 0.0).astype(jnp.float32)
    tri = jnp.where(
        lax.broadcasted_iota(jnp.int32, (_BLK, _BLK), 0)
        <= lax.broadcasted_iota(jnp.int32, (_BLK, _BLK), 1),
        1.0, 0.0).astype(jnp.float32)

    def row_from_col(v):
        return lax.dot_general(v, i256, (((0,), (0,)), ((), ())),
                               preferred_element_type=jnp.float32)

    def col_from_row(v):
        return lax.dot_general(i256, v, (((1,), (1,)), ((), ())),
                               preferred_element_type=jnp.float32)

    kept_ref[...] = jnp.zeros((1, C_PAD), jnp.float32)

    def block(t, _):
        base = t * _BLK
        y1b = bc_ref[pl.ds(base, _BLK), 0:1]
        x1b = bc_ref[pl.ds(base, _BLK), 1:2]
        y2b = bc_ref[pl.ds(base, _BLK), 2:3]
        x2b = bc_ref[pl.ds(base, _BLK), 3:4]
        area_b = jnp.maximum(y2b - y1b, 0.0) * jnp.maximum(x2b - x1b, 0.0)

        def chunk(c, sup):
            sl = pl.ds(c * _JCH, _JCH)
            y1c = br_ref[0:1, sl]
            x1c = br_ref[1:2, sl]
            y2c = br_ref[2:3, sl]
            x2c = br_ref[3:4, sl]
            area_c = br_ref[4:5, sl]
            kc = kept_ref[0:1, sl]
            yy1 = jnp.maximum(y1b, y1c)
            xx1 = jnp.maximum(x1b, x1c)
            yy2 = jnp.minimum(y2b, y2c)
            xx2 = jnp.minimum(x2b, x2c)
            inter = jnp.maximum(yy2 - yy1, 0.0) * jnp.maximum(xx2 - xx1, 0.0)
            union = area_b + area_c - inter
            ious = inter / (union + 1e-9)
            hit = jnp.where((ious > NMS_THRESHOLD) & (kc > 0.5), 1.0, 0.0)
            return jnp.maximum(sup, jnp.max(hit, axis=1, keepdims=True))

        sup_col = lax.fori_loop(0, _NJCH, chunk,
                                jnp.zeros((_BLK, 1), jnp.float32))

        sb = pl.ds(base, _BLK)
        y1s = br_ref[0:1, sb]
        x1s = br_ref[1:2, sb]
        y2s = br_ref[2:3, sb]
        x2s = br_ref[3:4, sb]
        area_s = br_ref[4:5, sb]
        yy1 = jnp.maximum(y1b, y1s)
        xx1 = jnp.maximum(x1b, x1s)
        yy2 = jnp.minimum(y2b, y2s)
        xx2 = jnp.minimum(x2b, x2s)
        inter = jnp.maximum(yy2 - yy1, 0.0) * jnp.maximum(xx2 - xx1, 0.0)
        union = area_b + area_s - inter
        ious = inter / (union + 1e-9)
        of = jnp.where(ious > NMS_THRESHOLD, 1.0, 0.0) * ltm

        jrow = base + lax.broadcasted_iota(jnp.int32, (1, _BLK), 1)
        invalid = jnp.where(jrow >= PRE_NMS, 1.0, 0.0)
        sup0 = jnp.maximum(row_from_col(sup_col), invalid)
        dec0 = jnp.zeros((1, _BLK), jnp.float32)

        def cond(state):
            sup, dec = state
            return jnp.sum((1.0 - sup) * (1.0 - dec)) > 0.5

        def wave(state):
            sup, dec = state
            und = (1.0 - sup) * (1.0 - dec)
            undc = col_from_row(und)
            blocked = jnp.max(of * undc, axis=0, keepdims=True)
            newk = und * (1.0 - blocked)
            dec = jnp.maximum(dec, newk)
            decc = col_from_row(dec)
            supn = jnp.max(of * decc, axis=0, keepdims=True)
            sup = jnp.maximum(sup, supn * (1.0 - dec))
            return (sup, dec)

        _, dec = lax.while_loop(cond, wave, (sup0, dec0))
        kept_ref[0:1, pl.ds(base, _BLK)] = dec
        return 0

    lax.fori_loop(0, _NBLK, block, 0)

    def csum(c, carry):
        kc = kept_ref[0:1, pl.ds(c * _BLK, _BLK)]
        inc = lax.dot_general(kc, tri, (((1,), (0,)), ((), ())),
                              preferred_element_type=jnp.float32)
        pos_ref[0:1, pl.ds(c * _BLK, _BLK)] = inc + carry
        return carry + jnp.sum(kc)

    nkept = lax.fori_loop(0, _NBLK, csum, jnp.float32(0.0))

    kept = kept_ref[...]
    pos_excl = pos_ref[...] - kept
    pio = lax.broadcasted_iota(jnp.int32, (OUT_PAD, 1), 0).astype(jnp.float32)
    jio = lax.broadcasted_iota(jnp.int32, (1, C_PAD), 1)
    eq = (kept > 0.5) & (pos_excl == pio)
    fb = (pio >= nkept) & (jio == 0)
    oh = jnp.where(eq | fb, 1.0, 0.0).astype(jnp.float32)
    res = lax.dot_general(oh, bc_ref[...], (((1,), (0,)), ((), ())),
                          precision=_HI, preferred_element_type=jnp.float32)
    scale = 1.0 / IMG_H
    z = jnp.zeros((OUT_PAD, 1), jnp.float32)
    out_ref[...] = jnp.concatenate(
        [res[:, 0:1] * scale, res[:, 1:2] * scale,
         res[:, 2:3] * scale, res[:, 3:4] * scale,
         res[:, 5:6], z, z, z, z, z, z, z, z, z, z, z], axis=1)


@functools.partial(jax.jit, static_argnames=("interpret",))
def _pipeline(rpn_pred_probs, rpn_box_deltas, anchors, interpret=False):
    scores = rpn_pred_probs[:, 1]
    pad = N_PAD - N_IN
    sp = jnp.concatenate([scores, jnp.full((pad,), -1.0, jnp.float32)])
    s2d = sp.reshape(_ROWS, 128)
    dz = jnp.zeros((pad, 4), jnp.float32)
    vals = jnp.concatenate(
        [jnp.concatenate([rpn_box_deltas, dz], axis=0),
         jnp.concatenate([anchors, dz], axis=0),
         sp.reshape(N_PAD, 1),
         jnp.zeros((N_PAD, 7), jnp.float32)], axis=1)

    msel, pose, bases_f = pl.pallas_call(
        _k1_select,
        out_shape=[jax.ShapeDtypeStruct((_ROWS, 128), jnp.float32),
                   jax.ShapeDtypeStruct((_ROWS, 128), jnp.float32),
                   jax.ShapeDtypeStruct((N_PAD // _KCH, 1), jnp.float32)],
        interpret=interpret,
    )(s2d)

    bases = ((bases_f.reshape(N_PAD // _KCH).astype(jnp.int32)) // 8) * 8

    bc = pl.pallas_call(
        _k2_gather_decode,
        out_shape=jax.ShapeDtypeStruct((C_PAD, 16), jnp.float32),
        in_specs=[pl.BlockSpec(memory_space=pltpu.SMEM),
                  pl.BlockSpec(memory_space=pltpu.VMEM),
                  pl.BlockSpec(memory_space=pltpu.VMEM),
                  pl.BlockSpec(memory_space=pltpu.VMEM)],
        scratch_shapes=[pltpu.VMEM((C_PAD + _WIN, 16), jnp.float32),
                        pltpu.VMEM((C_PAD, 16), jnp.float32),
                        pltpu.VMEM((1, C_PAD), jnp.float32),
                        pltpu.VMEM((1, C_PAD), jnp.float32)],
        interpret=interpret,
    )(bases, msel.reshape(1, N_PAD), pose.reshape(1, N_PAD), vals)

    br = bc.T

    res = pl.pallas_call(
        _k3_nms_assemble,
        out_shape=jax.ShapeDtypeStruct((OUT_PAD, 16), jnp.float32),
        scratch_shapes=[pltpu.VMEM((1, C_PAD), jnp.float32),
                        pltpu.VMEM((1, C_PAD), jnp.float32)],
        interpret=interpret,
    )(bc, br)

    return res[:PROPOSAL_COUNT, :5]


def kernel(rpn_pred_probs, rpn_box_deltas, anchors):
    return _pipeline(rpn_pred_probs, rpn_box_deltas, anchors)


# speedup vs baseline: 2.4333x; 1.4154x over previous
import functools

import jax
import jax.numpy as jnp
from jax import lax
from jax.experimental import pallas as pl
from jax.experimental.pallas import tpu as pltpu
from jax.experimental.pallas import tpu_sc as plsc

N_IN = 20000
N_PAD = 20480
PRE_NMS = 6000
C_PAD = 6144
D_PAD = 7168
PROPOSAL_COUNT = 1000
OUT_PAD = 1024
NMS_THRESHOLD = 0.7
IMG_H = 1024.0
IMG_W = 1024.0

_BLK = 256
_NBLK = C_PAD // _BLK
_JCH = 1024
_NJCH = C_PAD // _JCH
_ROWS = N_PAD // 128
_NW = 32

_HI = lax.Precision.HIGHEST


def _k1_select(s2d_ref, sidx_ref):
    u = lax.bitcast_convert_type(s2d_ref[...], jnp.int32)

    tri128 = jnp.where(
        lax.broadcasted_iota(jnp.int32, (128, 128), 0)
        <= lax.broadcasted_iota(jnp.int32, (128, 128), 1),
        1.0, 0.0).astype(jnp.float32)
    tris = jnp.where(
        lax.broadcasted_iota(jnp.int32, (_ROWS, _ROWS), 1)
        < lax.broadcasted_iota(jnp.int32, (_ROWS, _ROWS), 0),
        1.0, 0.0).astype(jnp.float32)

    def excl_cumsum(m):
        incl = lax.dot_general(m, tri128, (((1,), (0,)), ((), ())),
                               preferred_element_type=jnp.float32)
        rowsum = incl[:, 127:128]
        rowpref = lax.dot_general(tris, rowsum, (((1,), (0,)), ((), ())),
                                  preferred_element_type=jnp.float32)
        return incl + rowpref - m

    def bs(_, carry):
        lo, hi = carry
        mid = (lo + hi + 1) >> 1
        cnt = jnp.sum(jnp.where(u >= mid, 1.0, 0.0))
        ok = cnt >= float(PRE_NMS)
        return (jnp.where(ok, mid, lo), jnp.where(ok, hi, mid - 1))

    lo, _ = lax.fori_loop(0, 30, bs,
                          (jnp.int32(0), jnp.int32(0x3F800000)))
    t_val = lo
    gt = u > t_val
    n_gt = jnp.sum(jnp.where(gt, 1.0, 0.0))
    m_eq = jnp.where(u == t_val, 1.0, 0.0)
    quota = float(PRE_NMS) - n_gt
    eqex = excl_cumsum(m_eq)
    tie_sel = (m_eq > 0.5) & (eqex < quota)
    msel = jnp.where(gt | tie_sel, 1.0, 0.0)
    pose = excl_cumsum(msel).astype(jnp.int32)

    flat = (128 * lax.broadcasted_iota(jnp.int32, (_ROWS, 128), 0)
            + lax.broadcasted_iota(jnp.int32, (_ROWS, 128), 1))
    posu = flat - pose
    unsel = jnp.where(posu < C_PAD - PRE_NMS,
                      PRE_NMS + posu,
                      C_PAD + (posu & (D_PAD - C_PAD - 1)))
    sidx_ref[...] = jnp.where(msel > 0.5, pose, unsel)


def _make_sc_scatter(n_src, n_dst):
    n_per_w = n_src // _NW
    mesh = plsc.VectorSubcoreMesh(core_axis_name="c", subcore_axis_name="s")

    @functools.partial(
        pl.kernel,
        out_type=jax.ShapeDtypeStruct((n_dst, 128), jnp.float32),
        mesh=mesh,
        scratch_types=[pltpu.VMEM((n_per_w,), jnp.int32),
                       pltpu.VMEM((n_per_w, 128), jnp.float32),
                       pltpu.SemaphoreType.DMA],
    )
    def k(idx_hbm, rows_hbm, dst_hbm, idx_v, rows_v, sem):
        wid = lax.axis_index("s") * 2 + lax.axis_index("c")
        base = wid * n_per_w
        pltpu.sync_copy(idx_hbm.at[pl.ds(base, n_per_w)], idx_v)
        pltpu.sync_copy(rows_hbm.at[pl.ds(base, n_per_w)], rows_v)
        pltpu.async_copy(rows_v, dst_hbm.at[idx_v], sem).wait()

    return k


def _k2_decode_rank(cmp_ref, dec_ref, rankc_ref, srow_ref):
    acc = cmp_ref[...]

    d0 = acc[:, 0:1]
    d1 = acc[:, 1:2]
    d2 = acc[:, 2:3]
    d3 = acc[:, 3:4]
    ay1 = acc[:, 4:5]
    ax1 = acc[:, 5:6]
    ay2 = acc[:, 6:7]
    ax2 = acc[:, 7:8]
    sc = acc[:, 8:9]

    height = ay2 - ay1
    width = ax2 - ax1
    center_y = ay1 + 0.5 * height
    center_x = ax1 + 0.5 * width
    center_y = center_y + d0 * height
    center_x = center_x + d1 * width
    height = height * jnp.exp(d2)
    width = width * jnp.exp(d3)
    y1 = center_y - 0.5 * height
    x1 = center_x - 0.5 * width
    y2 = y1 + height
    x2 = x1 + width
    y1 = jnp.clip(y1, 0.0, IMG_H)
    x1 = jnp.clip(x1, 0.0, IMG_W)
    y2 = jnp.clip(y2, 0.0, IMG_H)
    x2 = jnp.clip(x2, 0.0, IMG_W)
    area = jnp.maximum(y2 - y1, 0.0) * jnp.maximum(x2 - x1, 0.0)

    z = jnp.zeros((C_PAD, 1), jnp.float32)
    dec_ref[...] = jnp.concatenate(
        [y1, x1, y2, x2, area, sc, z, z, z, z, z, z, z, z, z, z], axis=1)

    i256 = jnp.where(
        lax.broadcasted_iota(jnp.int32, (_BLK, _BLK), 0)
        == lax.broadcasted_iota(jnp.int32, (_BLK, _BLK), 1),
        1.0, 0.0).astype(jnp.float32)

    def row_from_col(v):
        return lax.dot_general(v, i256, (((0,), (0,)), ((), ())),
                               precision=_HI,
                               preferred_element_type=jnp.float32)

    def trow(t, _):
        scol = dec_ref[pl.ds(t * _BLK, _BLK), 5:6]
        srow_ref[0:1, pl.ds(t * _BLK, _BLK)] = row_from_col(scol)
        return 0

    lax.fori_loop(0, _NBLK, trow, 0)

    def rank_tile(t, _):
        si = dec_ref[pl.ds(t * _BLK, _BLK), 5:6]
        ii = t * _BLK + lax.broadcasted_iota(jnp.int32, (_BLK, 1), 0)

        def inner(c, cnt):
            sj = srow_ref[:, pl.ds(c * _JCH, _JCH)]
            jj = c * _JCH + lax.broadcasted_iota(jnp.int32, (1, _JCH), 1)
            cmp = (sj > si) | ((sj == si) & (jj < ii))
            return cnt + jnp.sum(jnp.where(cmp, 1.0, 0.0),
                                 axis=1, keepdims=True)

        cnt = lax.fori_loop(0, _NJCH, inner,
                            jnp.zeros((_BLK, 1), jnp.float32))
        rankc_ref[pl.ds(t * _BLK, _BLK), :] = cnt.astype(jnp.int32)
        return 0

    lax.fori_loop(0, _NBLK, rank_tile, 0)


def _k3_nms_assemble(bc_ref, br_ref, out_ref, kept_ref, pos_ref):
    i256 = jnp.where(
        lax.broadcasted_iota(jnp.int32, (_BLK, _BLK), 0)
        == lax.broadcasted_iota(jnp.int32, (_BLK, _BLK), 1),
        1.0, 0.0).astype(jnp.float32)
    ltm = jnp.where(
        lax.broadcasted_iota(jnp.int32, (_BLK, _BLK), 0)
        < lax.broadcasted_iota(jnp.int32, (_BLK, _BLK), 1),
        1.0, 0.0).astype(jnp.float32)
    tri = jnp.where(
        lax.broadcasted_iota(jnp.int32, (_BLK, _BLK), 0)
        <= lax.broadcasted_iota(jnp.int32, (_BLK, _BLK), 1),
        1.0, 0.0).astype(jnp.float32)

    def row_from_col(v):
        return lax.dot_general(v, i256, (((0,), (0,)), ((), ())),
                               preferred_element_type=jnp.float32)

    def col_from_row(v):
        return lax.dot_general(i256, v, (((1,), (1,)), ((), ())),
                               preferred_element_type=jnp.float32)

    kept_ref[...] = jnp.zeros((1, C_PAD), jnp.float32)

    def block(t, _):
        base = t * _BLK
        y1b = bc_ref[pl.ds(base, _BLK), 0:1]
        x1b = bc_ref[pl.ds(base, _BLK), 1:2]
        y2b = bc_ref[pl.ds(base, _BLK), 2:3]
        x2b = bc_ref[pl.ds(base, _BLK), 3:4]
        area_b = jnp.maximum(y2b - y1b, 0.0) * jnp.maximum(x2b - x1b, 0.0)

        def chunk(c, sup):
            sl = pl.ds(c * _JCH, _JCH)
            y1c = br_ref[0:1, sl]
            x1c = br_ref[1:2, sl]
            y2c = br_ref[2:3, sl]
            x2c = br_ref[3:4, sl]
            area_c = br_ref[4:5, sl]
            kc = kept_ref[0:1, sl]
            yy1 = jnp.maximum(y1b, y1c)
            xx1 = jnp.maximum(x1b, x1c)
            yy2 = jnp.minimum(y2b, y2c)
            xx2 = jnp.minimum(x2b, x2c)
            inter = jnp.maximum(yy2 - yy1, 0.0) * jnp.maximum(xx2 - xx1, 0.0)
            union = area_b + area_c - inter
            ious = inter / (union + 1e-9)
            hit = jnp.where((ious > NMS_THRESHOLD) & (kc > 0.5), 1.0, 0.0)
            return jnp.maximum(sup, jnp.max(hit, axis=1, keepdims=True))

        sup_col = lax.fori_loop(0, _NJCH, chunk,
                                jnp.zeros((_BLK, 1), jnp.float32))

        sb = pl.ds(base, _BLK)
        y1s = br_ref[0:1, sb]
        x1s = br_ref[1:2, sb]
        y2s = br_ref[2:3, sb]
        x2s = br_ref[3:4, sb]
        area_s = br_ref[4:5, sb]
        yy1 = jnp.maximum(y1b, y1s)
        xx1 = jnp.maximum(x1b, x1s)
        yy2 = jnp.minimum(y2b, y2s)
        xx2 = jnp.minimum(x2b, x2s)
        inter = jnp.maximum(yy2 - yy1, 0.0) * jnp.maximum(xx2 - xx1, 0.0)
        union = area_b + area_s - inter
        ious = inter / (union + 1e-9)
        of = jnp.where(ious > NMS_THRESHOLD, 1.0, 0.0) * ltm

        jrow = base + lax.broadcasted_iota(jnp.int32, (1, _BLK), 1)
        invalid = jnp.where(jrow >= PRE_NMS, 1.0, 0.0)
        sup0 = jnp.maximum(row_from_col(sup_col), invalid)
        dec0 = jnp.zeros((1, _BLK), jnp.float32)

        def cond(state):
            sup, dec = state
            return jnp.sum((1.0 - sup) * (1.0 - dec)) > 0.5

        def wave(state):
            sup, dec = state
            und = (1.0 - sup) * (1.0 - dec)
            undc = col_from_row(und)
            blocked = jnp.max(of * undc, axis=0, keepdims=True)
            newk = und * (1.0 - blocked)
            dec = jnp.maximum(dec, newk)
            decc = col_from_row(dec)
            supn = jnp.max(of * decc, axis=0, keepdims=True)
            sup = jnp.maximum(sup, supn * (1.0 - dec))
            return (sup, dec)

        _, dec = lax.while_loop(cond, wave, (sup0, dec0))
        kept_ref[0:1, pl.ds(base, _BLK)] = dec
        return 0

    lax.fori_loop(0, _NBLK, block, 0)

    def csum(c, carry):
        kc = kept_ref[0:1, pl.ds(c * _BLK, _BLK)]
        inc = lax.dot_general(kc, tri, (((1,), (0,)), ((), ())),
                              preferred_element_type=jnp.float32)
        pos_ref[0:1, pl.ds(c * _BLK, _BLK)] = inc + carry
        return carry + jnp.sum(kc)

    nkept = lax.fori_loop(0, _NBLK, csum, jnp.float32(0.0))

    kept = kept_ref[...]
    pos_excl = pos_ref[...] - kept
    pio = lax.broadcasted_iota(jnp.int32, (OUT_PAD, 1), 0).astype(jnp.float32)
    jio = lax.broadcasted_iota(jnp.int32, (1, C_PAD), 1)
    eq = (kept > 0.5) & (pos_excl == pio)
    fb = (pio >= nkept) & (jio == 0)
    oh = jnp.where(eq | fb, 1.0, 0.0).astype(jnp.float32)
    res = lax.dot_general(oh, bc_ref[...], (((1,), (0,)), ((), ())),
                          precision=_HI, preferred_element_type=jnp.float32)
    scale = 1.0 / IMG_H
    z = jnp.zeros((OUT_PAD, 1), jnp.float32)
    out_ref[...] = jnp.concatenate(
        [res[:, 0:1] * scale, res[:, 1:2] * scale,
         res[:, 2:3] * scale, res[:, 3:4] * scale,
         res[:, 5:6], z, z, z, z, z, z, z, z, z, z, z], axis=1)


@functools.partial(jax.jit, static_argnames=("interpret",))
def _pipeline(rpn_pred_probs, rpn_box_deltas, anchors, interpret=False):
    scores = rpn_pred_probs[:, 1]
    pad = N_PAD - N_IN
    sp = jnp.concatenate([scores, jnp.full((pad,), -1.0, jnp.float32)])
    s2d = sp.reshape(_ROWS, 128)
    dz = jnp.zeros((pad, 4), jnp.float32)
    vals = jnp.concatenate(
        [jnp.concatenate([rpn_box_deltas, dz], axis=0),
         jnp.concatenate([anchors, dz], axis=0),
         sp.reshape(N_PAD, 1),
         jnp.zeros((N_PAD, 119), jnp.float32)], axis=1)

    sidx = pl.pallas_call(
        _k1_select,
        out_shape=jax.ShapeDtypeStruct((_ROWS, 128), jnp.int32),
        interpret=interpret,
    )(s2d)

    cmp16 = _make_sc_scatter(N_PAD, D_PAD)(sidx.reshape(N_PAD), vals)
    cmp16 = cmp16[:C_PAD, :16]

    dec, rankc = pl.pallas_call(
        _k2_decode_rank,
        out_shape=[jax.ShapeDtypeStruct((C_PAD, 16), jnp.float32),
                   jax.ShapeDtypeStruct((C_PAD, 1), jnp.int32)],
        scratch_shapes=[pltpu.VMEM((1, C_PAD), jnp.float32)],
        interpret=interpret,
    )(cmp16)

    dec128 = jnp.concatenate(
        [dec, jnp.zeros((C_PAD, 112), jnp.float32)], axis=1)
    bc = _make_sc_scatter(C_PAD, C_PAD)(rankc.reshape(C_PAD), dec128)
    bc = bc[:, :16]

    br = bc.T

    res = pl.pallas_call(
        _k3_nms_assemble,
        out_shape=jax.ShapeDtypeStruct((OUT_PAD, 16), jnp.float32),
        scratch_shapes=[pltpu.VMEM((1, C_PAD), jnp.float32),
                        pltpu.VMEM((1, C_PAD), jnp.float32)],
        interpret=interpret,
    )(bc, br)

    return res[:PROPOSAL_COUNT, :5]


def kernel(rpn_pred_probs, rpn_box_deltas, anchors):
    return _pipeline(rpn_pred_probs, rpn_box_deltas, anchors)


# speedup vs baseline: 2.8773x; 1.1824x over previous
import functools

import jax
import jax.numpy as jnp
from jax import lax
from jax.experimental import pallas as pl
from jax.experimental.pallas import tpu as pltpu
from jax.experimental.pallas import tpu_sc as plsc

N_IN = 20000
N_PAD = 20480
PRE_NMS = 6000
C_PAD = 6144
D_PAD = 7168
PROPOSAL_COUNT = 1000
OUT_PAD = 1024
NMS_THRESHOLD = 0.7
IMG_H = 1024.0
IMG_W = 1024.0

_BLK = 256
_NBLK = C_PAD // _BLK
_JCH = 1024
_NJCH = C_PAD // _JCH
_ROWS = N_PAD // 128
_NW = 32

_HI = lax.Precision.HIGHEST


def _k1_select(s2d_ref, sidx_ref):
    u = lax.bitcast_convert_type(s2d_ref[...], jnp.int32)

    tri128 = jnp.where(
        lax.broadcasted_iota(jnp.int32, (128, 128), 0)
        <= lax.broadcasted_iota(jnp.int32, (128, 128), 1),
        1.0, 0.0).astype(jnp.float32)
    tris = jnp.where(
        lax.broadcasted_iota(jnp.int32, (_ROWS, _ROWS), 1)
        < lax.broadcasted_iota(jnp.int32, (_ROWS, _ROWS), 0),
        1.0, 0.0).astype(jnp.float32)

    def excl_cumsum(m):
        incl = lax.dot_general(m, tri128, (((1,), (0,)), ((), ())),
                               preferred_element_type=jnp.float32)
        rowsum = incl[:, 127:128]
        rowpref = lax.dot_general(tris, rowsum, (((1,), (0,)), ((), ())),
                                  preferred_element_type=jnp.float32)
        return incl + rowpref - m

    def bs(_, carry):
        lo, hi = carry
        mid = (lo + hi + 1) >> 1
        cnt = jnp.sum(jnp.where(u >= mid, 1.0, 0.0))
        ok = cnt >= float(PRE_NMS)
        return (jnp.where(ok, mid, lo), jnp.where(ok, hi, mid - 1))

    lo, _ = lax.fori_loop(0, 30, bs,
                          (jnp.int32(0), jnp.int32(0x3F800000)))
    t_val = lo
    gt = u > t_val
    n_gt = jnp.sum(jnp.where(gt, 1.0, 0.0))
    m_eq = jnp.where(u == t_val, 1.0, 0.0)
    quota = float(PRE_NMS) - n_gt
    eqex = excl_cumsum(m_eq)
    tie_sel = (m_eq > 0.5) & (eqex < quota)
    msel = jnp.where(gt | tie_sel, 1.0, 0.0)
    pose = excl_cumsum(msel).astype(jnp.int32)

    flat = (128 * lax.broadcasted_iota(jnp.int32, (_ROWS, 128), 0)
            + lax.broadcasted_iota(jnp.int32, (_ROWS, 128), 1))
    posu = flat - pose
    unsel = jnp.where(posu < C_PAD - PRE_NMS,
                      PRE_NMS + posu,
                      C_PAD + (posu & (D_PAD - C_PAD - 1)))
    sidx_ref[...] = jnp.where(msel > 0.5, pose, unsel)


def _make_sc_scatter(n_src, n_dst):
    n_per_w = n_src // _NW
    mesh = plsc.VectorSubcoreMesh(core_axis_name="c", subcore_axis_name="s")

    @functools.partial(
        pl.kernel,
        out_type=jax.ShapeDtypeStruct((n_dst, 128), jnp.float32),
        mesh=mesh,
        scratch_types=[pltpu.VMEM((n_per_w,), jnp.int32),
                       pltpu.VMEM((n_per_w, 128), jnp.float32),
                       pltpu.SemaphoreType.DMA],
    )
    def k(idx_hbm, rows_hbm, dst_hbm, idx_v, rows_v, sem):
        wid = lax.axis_index("s") * 2 + lax.axis_index("c")
        base = wid * n_per_w
        pltpu.sync_copy(idx_hbm.at[pl.ds(base, n_per_w)], idx_v)
        pltpu.sync_copy(rows_hbm.at[pl.ds(base, n_per_w)], rows_v)
        pltpu.async_copy(rows_v, dst_hbm.at[idx_v], sem).wait()

    return k


def _k2_decode_rank(cmp_ref, dec_ref, rankc_ref, srow_ref):
    acc = cmp_ref[...]

    d0 = acc[:, 0:1]
    d1 = acc[:, 1:2]
    d2 = acc[:, 2:3]
    d3 = acc[:, 3:4]
    ay1 = acc[:, 4:5]
    ax1 = acc[:, 5:6]
    ay2 = acc[:, 6:7]
    ax2 = acc[:, 7:8]
    sc = acc[:, 8:9]

    height = ay2 - ay1
    width = ax2 - ax1
    center_y = ay1 + 0.5 * height
    center_x = ax1 + 0.5 * width
    center_y = center_y + d0 * height
    center_x = center_x + d1 * width
    height = height * jnp.exp(d2)
    width = width * jnp.exp(d3)
    y1 = center_y - 0.5 * height
    x1 = center_x - 0.5 * width
    y2 = y1 + height
    x2 = x1 + width
    y1 = jnp.clip(y1, 0.0, IMG_H)
    x1 = jnp.clip(x1, 0.0, IMG_W)
    y2 = jnp.clip(y2, 0.0, IMG_H)
    x2 = jnp.clip(x2, 0.0, IMG_W)
    area = jnp.maximum(y2 - y1, 0.0) * jnp.maximum(x2 - x1, 0.0)

    z = jnp.zeros((C_PAD, 1), jnp.float32)
    dec_ref[...] = jnp.concatenate(
        [y1, x1, y2, x2, area, sc, z, z, z, z, z, z, z, z, z, z], axis=1)

    i256 = jnp.where(
        lax.broadcasted_iota(jnp.int32, (_BLK, _BLK), 0)
        == lax.broadcasted_iota(jnp.int32, (_BLK, _BLK), 1),
        1.0, 0.0).astype(jnp.float32)

    def row_from_col(v):
        return lax.dot_general(v, i256, (((0,), (0,)), ((), ())),
                               precision=_HI,
                               preferred_element_type=jnp.float32)

    def trow(t, _):
        scol = dec_ref[pl.ds(t * _BLK, _BLK), 5:6]
        srow_ref[0:1, pl.ds(t * _BLK, _BLK)] = row_from_col(scol)
        return 0

    lax.fori_loop(0, _NBLK, trow, 0)

    def rank_tile(t, _):
        si = dec_ref[pl.ds(t * _BLK, _BLK), 5:6]
        ii = t * _BLK + lax.broadcasted_iota(jnp.int32, (_BLK, 1), 0)

        def inner(c, cnt):
            sj = srow_ref[:, pl.ds(c * _JCH, _JCH)]
            jj = c * _JCH + lax.broadcasted_iota(jnp.int32, (1, _JCH), 1)
            cmp = (sj > si) | ((sj == si) & (jj < ii))
            return cnt + jnp.sum(jnp.where(cmp, 1.0, 0.0),
                                 axis=1, keepdims=True)

        cnt = lax.fori_loop(0, _NJCH, inner,
                            jnp.zeros((_BLK, 1), jnp.float32))
        rankc_ref[pl.ds(t * _BLK, _BLK), :] = cnt.astype(jnp.int32)
        return 0

    lax.fori_loop(0, _NBLK, rank_tile, 0)


def _k3_nms_assemble(bc_ref, br_ref, out_ref, kept_ref, pos_ref):
    i256 = jnp.where(
        lax.broadcasted_iota(jnp.int32, (_BLK, _BLK), 0)
        == lax.broadcasted_iota(jnp.int32, (_BLK, _BLK), 1),
        1.0, 0.0).astype(jnp.float32)
    ltm = jnp.where(
        lax.broadcasted_iota(jnp.int32, (_BLK, _BLK), 0)
        < lax.broadcasted_iota(jnp.int32, (_BLK, _BLK), 1),
        1.0, 0.0).astype(jnp.float32)
    tri = jnp.where(
        lax.broadcasted_iota(jnp.int32, (_BLK, _BLK), 0)
        <= lax.broadcasted_iota(jnp.int32, (_BLK, _BLK), 1),
        1.0, 0.0).astype(jnp.float32)

    def row_from_col(v):
        return lax.dot_general(v, i256, (((0,), (0,)), ((), ())),
                               preferred_element_type=jnp.float32)

    def col_from_row(v):
        return lax.dot_general(i256, v, (((1,), (1,)), ((), ())),
                               preferred_element_type=jnp.float32)

    kept_ref[...] = jnp.zeros((1, C_PAD), jnp.float32)

    def block(t, _):
        base = t * _BLK
        y1b = bc_ref[pl.ds(base, _BLK), 0:1]
        x1b = bc_ref[pl.ds(base, _BLK), 1:2]
        y2b = bc_ref[pl.ds(base, _BLK), 2:3]
        x2b = bc_ref[pl.ds(base, _BLK), 3:4]
        area_b = jnp.maximum(y2b - y1b, 0.0) * jnp.maximum(x2b - x1b, 0.0)

        def chunk(c, sup):
            sl = pl.ds(c * _JCH, _JCH)
            y1c = br_ref[0:1, sl]
            x1c = br_ref[1:2, sl]
            y2c = br_ref[2:3, sl]
            x2c = br_ref[3:4, sl]
            area_c = br_ref[4:5, sl]
            kc = kept_ref[0:1, sl]
            yy1 = jnp.maximum(y1b, y1c)
            xx1 = jnp.maximum(x1b, x1c)
            yy2 = jnp.minimum(y2b, y2c)
            xx2 = jnp.minimum(x2b, x2c)
            inter = jnp.maximum(yy2 - yy1, 0.0) * jnp.maximum(xx2 - xx1, 0.0)
            union = area_b + area_c - inter
            ious = inter / (union + 1e-9)
            hit = jnp.where((ious > NMS_THRESHOLD) & (kc > 0.5), 1.0, 0.0)
            return jnp.maximum(sup, jnp.max(hit, axis=1, keepdims=True))

        sup_col = lax.fori_loop(0, t // (_JCH // _BLK) + 1, chunk,
                                jnp.zeros((_BLK, 1), jnp.float32))

        sb = pl.ds(base, _BLK)
        y1s = br_ref[0:1, sb]
        x1s = br_ref[1:2, sb]
        y2s = br_ref[2:3, sb]
        x2s = br_ref[3:4, sb]
        area_s = br_ref[4:5, sb]
        yy1 = jnp.maximum(y1b, y1s)
        xx1 = jnp.maximum(x1b, x1s)
        yy2 = jnp.minimum(y2b, y2s)
        xx2 = jnp.minimum(x2b, x2s)
        inter = jnp.maximum(yy2 - yy1, 0.0) * jnp.maximum(xx2 - xx1, 0.0)
        union = area_b + area_s - inter
        ious = inter / (union + 1e-9)
        of = jnp.where(ious > NMS_THRESHOLD, 1.0, 0.0) * ltm

        jrow = base + lax.broadcasted_iota(jnp.int32, (1, _BLK), 1)
        invalid = jnp.where(jrow >= PRE_NMS, 1.0, 0.0)
        sup0 = jnp.maximum(row_from_col(sup_col), invalid)
        dec0 = jnp.zeros((1, _BLK), jnp.float32)

        def cond(state):
            sup, dec = state
            return jnp.sum((1.0 - sup) * (1.0 - dec)) > 0.5

        def wave(state):
            sup, dec = state
            und = (1.0 - sup) * (1.0 - dec)
            undc = col_from_row(und)
            blocked = jnp.max(of * undc, axis=0, keepdims=True)
            newk = und * (1.0 - blocked)
            dec = jnp.maximum(dec, newk)
            decc = col_from_row(dec)
            supn = jnp.max(of * decc, axis=0, keepdims=True)
            sup = jnp.maximum(sup, supn * (1.0 - dec))
            return (sup, dec)

        _, dec = lax.while_loop(cond, wave, (sup0, dec0))
        kept_ref[0:1, pl.ds(base, _BLK)] = dec
        return 0

    lax.fori_loop(0, _NBLK, block, 0)

    def csum(c, carry):
        kc = kept_ref[0:1, pl.ds(c * _BLK, _BLK)]
        inc = lax.dot_general(kc, tri, (((1,), (0,)), ((), ())),
                              preferred_element_type=jnp.float32)
        pos_ref[0:1, pl.ds(c * _BLK, _BLK)] = inc + carry
        return carry + jnp.sum(kc)

    nkept = lax.fori_loop(0, _NBLK, csum, jnp.float32(0.0))

    kept = kept_ref[...]
    pos_excl = pos_ref[...] - kept
    pio = lax.broadcasted_iota(jnp.int32, (OUT_PAD, 1), 0).astype(jnp.float32)
    jio = lax.broadcasted_iota(jnp.int32, (1, C_PAD), 1)
    eq = (kept > 0.5) & (pos_excl == pio)
    fb = (pio >= nkept) & (jio == 0)
    oh = jnp.where(eq | fb, 1.0, 0.0).astype(jnp.float32)
    res = lax.dot_general(oh, bc_ref[...], (((1,), (0,)), ((), ())),
                          precision=_HI, preferred_element_type=jnp.float32)
    scale = 1.0 / IMG_H
    z = jnp.zeros((OUT_PAD, 1), jnp.float32)
    out_ref[...] = jnp.concatenate(
        [res[:, 0:1] * scale, res[:, 1:2] * scale,
         res[:, 2:3] * scale, res[:, 3:4] * scale,
         res[:, 5:6], z, z, z, z, z, z, z, z, z, z, z], axis=1)


@functools.partial(jax.jit, static_argnames=("interpret",))
def _pipeline(rpn_pred_probs, rpn_box_deltas, anchors, interpret=False):
    scores = rpn_pred_probs[:, 1]
    pad = N_PAD - N_IN
    sp = jnp.concatenate([scores, jnp.full((pad,), -1.0, jnp.float32)])
    s2d = sp.reshape(_ROWS, 128)
    dz = jnp.zeros((pad, 4), jnp.float32)
    vals = jnp.concatenate(
        [jnp.concatenate([rpn_box_deltas, dz], axis=0),
         jnp.concatenate([anchors, dz], axis=0),
         sp.reshape(N_PAD, 1),
         jnp.zeros((N_PAD, 119), jnp.float32)], axis=1)

    sidx = pl.pallas_call(
        _k1_select,
        out_shape=jax.ShapeDtypeStruct((_ROWS, 128), jnp.int32),
        interpret=interpret,
    )(s2d)

    cmp16 = _make_sc_scatter(N_PAD, D_PAD)(sidx.reshape(N_PAD), vals)
    cmp16 = cmp16[:C_PAD, :16]

    dec, rankc = pl.pallas_call(
        _k2_decode_rank,
        out_shape=[jax.ShapeDtypeStruct((C_PAD, 16), jnp.float32),
                   jax.ShapeDtypeStruct((C_PAD, 1), jnp.int32)],
        scratch_shapes=[pltpu.VMEM((1, C_PAD), jnp.float32)],
        interpret=interpret,
    )(cmp16)

    dec128 = jnp.concatenate(
        [dec, jnp.zeros((C_PAD, 112), jnp.float32)], axis=1)
    bc = _make_sc_scatter(C_PAD, C_PAD)(rankc.reshape(C_PAD), dec128)
    bc = bc[:, :16]

    br = bc.T

    res = pl.pallas_call(
        _k3_nms_assemble,
        out_shape=jax.ShapeDtypeStruct((OUT_PAD, 16), jnp.float32),
        scratch_shapes=[pltpu.VMEM((1, C_PAD), jnp.float32),
                        pltpu.VMEM((1, C_PAD), jnp.float32)],
        interpret=interpret,
    )(bc, br)

    return res[:PROPOSAL_COUNT, :5]


def kernel(rpn_pred_probs, rpn_box_deltas, anchors):
    return _pipeline(rpn_pred_probs, rpn_box_deltas, anchors)
